# baseline (speedup 1.0000x reference)
import jax
import jax.numpy as jnp
from jax.experimental import pallas as pl

_ROWS = 10
_COLS = 100000
_BLOCK_W = 100000

_KEEP_ROWS = (3, 5, 8)
_VAL = -0.281894713640213


def _select_body(x_ref, o_ref):
    ri = jax.lax.broadcasted_iota(jnp.int32, (_ROWS, _BLOCK_W), 0)
    keep = (ri == _KEEP_ROWS[0]) | (ri == _KEEP_ROWS[1]) | (ri == _KEEP_ROWS[2])
    o_ref[...] = jnp.where(keep, x_ref[...], jnp.float32(_VAL))


def kernel(x):
    grid = (pl.cdiv(_COLS, _BLOCK_W),)
    return pl.pallas_call(
        _select_body,
        grid=grid,
        in_specs=[pl.BlockSpec((_ROWS, _BLOCK_W), lambda i: (0, i))],
        out_specs=pl.BlockSpec((_ROWS, _BLOCK_W), lambda i: (0, i)),
        out_shape=jax.ShapeDtypeStruct((_ROWS, _COLS), jnp.float32),
    )(x)


# speedup vs baseline: 1.0057x; 1.0057x over previous
import jax
import jax.numpy as jnp
from jax.experimental import pallas as pl

_ROWS = 10
_COLS = 100000
_BLOCK_W = 38400

_KEEP_ROWS = (3, 5, 8)
_VAL = -0.281894713640213


def _select_body(x_ref, o_ref):
    ri = jax.lax.broadcasted_iota(jnp.int32, (_ROWS, _BLOCK_W), 0)
    keep = (ri == _KEEP_ROWS[0]) | (ri == _KEEP_ROWS[1]) | (ri == _KEEP_ROWS[2])
    o_ref[...] = jnp.where(keep, x_ref[...], jnp.float32(_VAL))


def kernel(x):
    grid = (pl.cdiv(_COLS, _BLOCK_W),)
    return pl.pallas_call(
        _select_body,
        grid=grid,
        in_specs=[pl.BlockSpec((_ROWS, _BLOCK_W), lambda i: (0, i))],
        out_specs=pl.BlockSpec((_ROWS, _BLOCK_W), lambda i: (0, i)),
        out_shape=jax.ShapeDtypeStruct((_ROWS, _COLS), jnp.float32),
    )(x)


# speedup vs baseline: 1.2044x; 1.1976x over previous
import jax
import jax.numpy as jnp
from jax.experimental import pallas as pl

_ROWS = 10
_COLS = 100000
_BLOCK_W = 51200

_KEEP_ROWS = (3, 5, 8)
_VAL = -0.281894713640213


def _select_body(x_ref, o_ref):
    ri = jax.lax.broadcasted_iota(jnp.int32, (_ROWS, _BLOCK_W), 0)
    keep = (ri == _KEEP_ROWS[0]) | (ri == _KEEP_ROWS[1]) | (ri == _KEEP_ROWS[2])
    o_ref[...] = jnp.where(keep, x_ref[...], jnp.float32(_VAL))


def kernel(x):
    grid = (pl.cdiv(_COLS, _BLOCK_W),)
    return pl.pallas_call(
        _select_body,
        grid=grid,
        in_specs=[pl.BlockSpec((_ROWS, _BLOCK_W), lambda i: (0, i))],
        out_specs=pl.BlockSpec((_ROWS, _BLOCK_W), lambda i: (0, i)),
        out_shape=jax.ShapeDtypeStruct((_ROWS, _COLS), jnp.float32),
    )(x)


# speedup vs baseline: 1.2381x; 1.0280x over previous
import jax
import jax.numpy as jnp
from jax.experimental import pallas as pl

_ROWS = 10
_COLS = 100000
_BLOCK_W = 50048

_KEEP_ROWS = (3, 5, 8)
_VAL = -0.281894713640213


def _select_body(x_ref, o_ref):
    ri = jax.lax.broadcasted_iota(jnp.int32, (_ROWS, _BLOCK_W), 0)
    keep = (ri == _KEEP_ROWS[0]) | (ri == _KEEP_ROWS[1]) | (ri == _KEEP_ROWS[2])
    o_ref[...] = jnp.where(keep, x_ref[...], jnp.float32(_VAL))


def kernel(x):
    grid = (pl.cdiv(_COLS, _BLOCK_W),)
    return pl.pallas_call(
        _select_body,
        grid=grid,
        in_specs=[pl.BlockSpec((_ROWS, _BLOCK_W), lambda i: (0, i))],
        out_specs=pl.BlockSpec((_ROWS, _BLOCK_W), lambda i: (0, i)),
        out_shape=jax.ShapeDtypeStruct((_ROWS, _COLS), jnp.float32),
    )(x)
